# baseline (speedup 1.0000x reference)
import functools

import jax
import jax.numpy as jnp
from jax import lax
from jax.experimental import pallas as pl
from jax.experimental.pallas import tpu as pltpu
from jax.experimental.pallas import tpu_sc as plsc

_B = 4096
_N = 1000
_NPAD = 1008
_LANES = 16
_CHUNK = 16
_IN_W = _CHUNK * _N
_OUT_W = _CHUNK * _N + 16


def _remap_body(nc, rows_per_w, x_hbm, map_hbm, out_hbm,
                map_v, in_buf, out_buf,
                sem_in0, sem_in1, sem_out0, sem_out1):
    cid = lax.axis_index("c")
    sid = lax.axis_index("s")
    wid = sid * nc + cid
    base0 = wid * rows_per_w
    nchunks = rows_per_w // _CHUNK

    pltpu.sync_copy(map_hbm, map_v)

    sem_in = (sem_in0, sem_in1)
    sem_out = (sem_out0, sem_out1)

    def start_in(g):
        ph = g % 2
        return pltpu.async_copy(
            x_hbm.at[pl.ds((base0 + g * _CHUNK) * _N, _IN_W)],
            in_buf.at[pl.ds(ph * _IN_W, _IN_W)],
            sem_in[ph])

    def start_out(g):
        ph = g % 2
        return pltpu.async_copy(
            out_buf.at[pl.ds(ph * _OUT_W, _IN_W)],
            out_hbm.at[pl.ds((base0 + g * _CHUNK) * _N, _IN_W)],
            sem_out[ph])

    def gather_store(ph, r, idx_raw, out_off):
        idxc = jnp.minimum(idx_raw, _N - 1) + (ph * _IN_W + r * _N)
        vals = plsc.load_gather(in_buf, [idxc])
        vals = jnp.where(idx_raw < _N, vals, 0.0)
        out_buf[pl.ds(out_off, _LANES)] = vals

    pending_in = {0: start_in(0)}
    pending_out = {}

    for g in range(nchunks):
        ph = g % 2
        if g + 1 < nchunks:
            pending_in[g + 1] = start_in(g + 1)
        pending_in.pop(g).wait()
        if g - 2 in pending_out:
            pending_out.pop(g - 2).wait()

        idx_tail = map_v[pl.ds(_NPAD - _LANES, _LANES)]
        for r in range(_CHUNK):
            gather_store(ph, r, idx_tail,
                         ph * _OUT_W + r * _N + (_NPAD - _LANES))

        def kstep(k, _):
            idxk = map_v[pl.ds(k * _LANES, _LANES)]
            for r in range(_CHUNK):
                gather_store(ph, r, idxk,
                             ph * _OUT_W + r * _N + k * _LANES)
            return 0
        lax.fori_loop(0, _NPAD // _LANES - 1, kstep, 0, unroll=2)

        pending_out[g] = start_out(g)

    for g in sorted(pending_out):
        pending_out.pop(g).wait()


def kernel(x, mapping):
    mapping = jnp.concatenate(
        [mapping.astype(jnp.int32),
         jnp.full((_NPAD - _N,), _N, jnp.int32)])

    info = plsc.get_sparse_core_info()
    nw = info.num_cores * info.num_subcores
    rows_per_w = _B // nw

    mesh = plsc.VectorSubcoreMesh(core_axis_name="c", subcore_axis_name="s")
    f = pl.kernel(
        functools.partial(_remap_body, info.num_cores, rows_per_w),
        out_type=jax.ShapeDtypeStruct((_B * _N,), jnp.float32),
        mesh=mesh,
        compiler_params=pltpu.CompilerParams(needs_layout_passes=False),
        scratch_types=[
            pltpu.VMEM((_NPAD,), jnp.int32),
            pltpu.VMEM((2 * _IN_W,), jnp.float32),
            pltpu.VMEM((2 * _OUT_W,), jnp.float32),
            pltpu.SemaphoreType.DMA,
            pltpu.SemaphoreType.DMA,
            pltpu.SemaphoreType.DMA,
            pltpu.SemaphoreType.DMA,
        ],
    )
    return f(x.reshape(-1), mapping).reshape(_B, _N)


# speedup vs baseline: 1.2092x; 1.2092x over previous
import functools

import jax
import jax.numpy as jnp
from jax import lax
from jax.experimental import pallas as pl
from jax.experimental.pallas import tpu as pltpu
from jax.experimental.pallas import tpu_sc as plsc

_B = 4096
_N = 1000
_NPAD = 1008
_LANES = 16
_CHUNK = 16
_IN_W = _CHUNK * _N
_OUT_W = _CHUNK * _N + 16


def _remap_body(nc, rows_per_w, x_hbm, map_hbm, out_hbm,
                map_v, in_buf, out_buf,
                sem_in0, sem_in1, sem_out0, sem_out1):
    cid = lax.axis_index("c")
    sid = lax.axis_index("s")
    wid = sid * nc + cid
    base0 = wid * rows_per_w
    nchunks = rows_per_w // _CHUNK

    pltpu.sync_copy(map_hbm, map_v)

    sem_in = (sem_in0, sem_in1)
    sem_out = (sem_out0, sem_out1)

    def start_in(g):
        ph = g % 2
        return pltpu.async_copy(
            x_hbm.at[pl.ds((base0 + g * _CHUNK) * _N, _IN_W)],
            in_buf.at[pl.ds(ph * _IN_W, _IN_W)],
            sem_in[ph])

    def start_out(g):
        ph = g % 2
        return pltpu.async_copy(
            out_buf.at[pl.ds(ph * _OUT_W, _IN_W)],
            out_hbm.at[pl.ds((base0 + g * _CHUNK) * _N, _IN_W)],
            sem_out[ph])

    ntail = _N - (_N // _LANES) * _LANES
    tail_mask = jnp.arange(_LANES, dtype=jnp.int32) < ntail

    def gather(ph, r, idxc, ok):
        vals = plsc.load_gather(in_buf, [idxc + (ph * _IN_W + r * _N)])
        return jnp.where(ok, vals, 0.0)

    def compute_chunk(ph):
        @plsc.parallel_loop(0, _N // _LANES, unroll=4)
        def _(k):
            idxk = map_v[pl.ds(k * _LANES, _LANES)]
            idxc = jnp.minimum(idxk, _N - 1)
            ok = idxk < _N
            for r in range(_CHUNK):
                out_buf[pl.ds(ph * _OUT_W + r * _N + k * _LANES, _LANES)] = (
                    gather(ph, r, idxc, ok))

        idx_tail = map_v[pl.ds(_NPAD - _LANES, _LANES)]
        idxc = jnp.minimum(idx_tail, _N - 1)
        ok = idx_tail < _N
        for r in range(_CHUNK):
            plsc.store_compressed(
                out_buf.at[pl.ds(ph * _OUT_W + (r + 1) * _N - ntail, _LANES)],
                gather(ph, r, idxc, ok), mask=tail_mask)

    pending_in = {0: start_in(0)}
    pending_out = {}

    for g in range(nchunks):
        ph = g % 2
        if g + 1 < nchunks:
            pending_in[g + 1] = start_in(g + 1)
        pending_in.pop(g).wait()
        if g - 2 in pending_out:
            pending_out.pop(g - 2).wait()
        compute_chunk(ph)
        pending_out[g] = start_out(g)

    for g in sorted(pending_out):
        pending_out.pop(g).wait()


def kernel(x, mapping):
    mapping = jnp.concatenate(
        [mapping.astype(jnp.int32),
         jnp.full((_NPAD - _N,), _N, jnp.int32)])

    info = plsc.get_sparse_core_info()
    nw = info.num_cores * info.num_subcores
    rows_per_w = _B // nw

    mesh = plsc.VectorSubcoreMesh(core_axis_name="c", subcore_axis_name="s")
    f = pl.kernel(
        functools.partial(_remap_body, info.num_cores, rows_per_w),
        out_type=jax.ShapeDtypeStruct((_B * _N,), jnp.float32),
        mesh=mesh,
        compiler_params=pltpu.CompilerParams(needs_layout_passes=False),
        scratch_types=[
            pltpu.VMEM((_NPAD,), jnp.int32),
            pltpu.VMEM((2 * _IN_W,), jnp.float32),
            pltpu.VMEM((2 * _OUT_W,), jnp.float32),
            pltpu.SemaphoreType.DMA,
            pltpu.SemaphoreType.DMA,
            pltpu.SemaphoreType.DMA,
            pltpu.SemaphoreType.DMA,
        ],
    )
    return f(x.reshape(-1), mapping).reshape(_B, _N)


# speedup vs baseline: 1.8091x; 1.4961x over previous
import functools

import jax
import jax.numpy as jnp
from jax import lax
from jax.experimental import pallas as pl
from jax.experimental.pallas import tpu as pltpu
from jax.experimental.pallas import tpu_sc as plsc

_B = 4096
_N = 1000
_LANES = 16
_CHUNK = 16
_NFULL = _N // _LANES
_TAIL_OFF = _N - _LANES


def _remap_body(nc, rows_per_w, x_hbm, map_hbm, out_hbm,
                map_v, in_buf, out_buf,
                sem_in0, sem_in1, sem_out0, sem_out1):
    cid = lax.axis_index("c")
    sid = lax.axis_index("s")
    wid = sid * nc + cid
    base0 = wid * rows_per_w
    nchunks = rows_per_w // _CHUNK

    pltpu.sync_copy(map_hbm, map_v)

    sem_in = (sem_in0, sem_in1)
    sem_out = (sem_out0, sem_out1)

    def start_in(g):
        return pltpu.async_copy(
            x_hbm.at[pl.ds(base0 + g * _CHUNK, _CHUNK)],
            in_buf.at[g % 2], sem_in[g % 2])

    def start_out(g):
        return pltpu.async_copy(
            out_buf.at[g % 2],
            out_hbm.at[pl.ds(base0 + g * _CHUNK, _CHUNK)], sem_out[g % 2])

    row_ids = [jnp.full((_LANES,), r, jnp.int32) for r in range(_CHUNK)]

    def block(ph, col_off, idxk):
        idxc = jnp.minimum(idxk, _N - 1)
        ok = idxk < _N
        for r in range(_CHUNK):
            vals = plsc.load_gather(in_buf.at[ph], [row_ids[r], idxc])
            out_buf[ph, r, pl.ds(col_off, _LANES)] = jnp.where(ok, vals, 0.0)

    def compute_chunk(ph):
        @plsc.parallel_loop(0, _NFULL, unroll=4)
        def _(k):
            block(ph, k * _LANES, map_v[pl.ds(k * _LANES, _LANES)])
        block(ph, _TAIL_OFF, map_v[pl.ds(_TAIL_OFF, _LANES)])

    pending_in = {0: start_in(0)}
    pending_out = {}

    for g in range(nchunks):
        ph = g % 2
        if g + 1 < nchunks:
            pending_in[g + 1] = start_in(g + 1)
        pending_in.pop(g).wait()
        if g - 2 in pending_out:
            pending_out.pop(g - 2).wait()
        compute_chunk(ph)
        pending_out[g] = start_out(g)

    for g in sorted(pending_out):
        pending_out.pop(g).wait()


def kernel(x, mapping):
    mapping = mapping.astype(jnp.int32)

    info = plsc.get_sparse_core_info()
    nw = info.num_cores * info.num_subcores
    rows_per_w = _B // nw

    mesh = plsc.VectorSubcoreMesh(core_axis_name="c", subcore_axis_name="s")
    f = pl.kernel(
        functools.partial(_remap_body, info.num_cores, rows_per_w),
        out_type=jax.ShapeDtypeStruct((_B, _N), jnp.float32),
        mesh=mesh,
        compiler_params=pltpu.CompilerParams(needs_layout_passes=False),
        scratch_types=[
            pltpu.VMEM((_N,), jnp.int32),
            pltpu.VMEM((2, _CHUNK, _N), jnp.float32),
            pltpu.VMEM((2, _CHUNK, _N), jnp.float32),
            pltpu.SemaphoreType.DMA,
            pltpu.SemaphoreType.DMA,
            pltpu.SemaphoreType.DMA,
            pltpu.SemaphoreType.DMA,
        ],
    )
    return f(x, mapping)
